# baseline (speedup 1.0000x reference)
import functools

import jax
import jax.numpy as jnp
from jax import lax
from jax.experimental import pallas as pl
from jax.experimental.pallas import tpu as pltpu
from jax.experimental.pallas import tpu_sc as plsc

_BATCH = 4096
_FIELDS = 26
_K = 32
_NC = 2
_NS = 16
_NW = _NC * _NS
_ROWS = _BATCH * _FIELDS
_RPW = _ROWS // _NW
_CH = 128
_NCH = _RPW // _CH


@functools.partial(
    pl.kernel,
    out_type=jax.ShapeDtypeStruct((_ROWS, _K), jnp.float32),
    mesh=plsc.VectorSubcoreMesh(core_axis_name="c", subcore_axis_name="s"),
    scratch_types=[
        pltpu.VMEM((_NW * _NCH // _NW, _CH), jnp.int32),
        pltpu.VMEM((_RPW,), jnp.float32),
        pltpu.VMEM((_RPW, _K), jnp.float32),
        pltpu.SemaphoreType.DMA,
    ],
    compiler_params=pltpu.CompilerParams(use_tc_tiling_on_sc=False),
)
def _gather_scale(idx_hbm, val_hbm, table_hbm, out_hbm, idx_v, val_v, rows_v, sem):
    wid = lax.axis_index("s") * _NC + lax.axis_index("c")
    base = wid * _RPW
    pltpu.sync_copy(idx_hbm.at[wid], idx_v)
    pltpu.sync_copy(val_hbm.at[pl.ds(base, _RPW)], val_v)
    descs = [
        pltpu.async_copy(
            table_hbm.at[idx_v.at[j]], rows_v.at[pl.ds(j * _CH, _CH)], sem
        )
        for j in range(_NCH)
    ]
    for d in descs:
        d.wait()

    def body(g, carry):
        r0 = g * 16
        vals = val_v[pl.ds(r0, 16)]
        for j in range(16):
            s = vals[j]
            rows_v[r0 + j, pl.ds(0, 16)] = rows_v[r0 + j, pl.ds(0, 16)] * s
            rows_v[r0 + j, pl.ds(16, 16)] = rows_v[r0 + j, pl.ds(16, 16)] * s
        return carry

    lax.fori_loop(0, _RPW // 16, body, 0)
    pltpu.sync_copy(rows_v, out_hbm.at[pl.ds(base, _RPW)])


def kernel(nonzero_index, nonzero_value, table):
    idx = nonzero_index.reshape(_NW, _NCH, _CH).astype(jnp.int32)
    val = nonzero_value.reshape(_ROWS)
    out = _gather_scale(idx, val, table)
    return out.reshape(_BATCH, _FIELDS, _K)
